# baseline (speedup 1.0000x reference)
import jax
import jax.numpy as jnp
from jax import lax
from jax.experimental import pallas as pl
from jax.experimental.pallas import tpu as pltpu
from jax.experimental.pallas import tpu_sc as plsc

B, S, K = 4, 2048, 768
N_CLASSES = 42
NC, NS, L = 2, 16, 16
NW = NC * NS
NFC = NW // B
FC = K // NFC
NV = FC // L
TCHUNK = 512
NCHUNK = S // TCHUNK


def _pool_body(x_hbm, m1_hbm, m2_hbm, out1_hbm, out2_hbm,
               xbuf, m1buf, m2buf, res1, res2):
    wid = lax.axis_index("s") * NC + lax.axis_index("c")
    bi = wid // NFC
    fc = wid % NFC
    f0 = fc * FC

    pltpu.sync_copy(m1_hbm.at[bi], m1buf)
    pltpu.sync_copy(m2_hbm.at[bi], m2buf)

    neg = jnp.full((L,), -jnp.inf, jnp.float32)
    accs = [neg] * (2 * NV)

    def chunk_body(ci, accs):
        pltpu.sync_copy(x_hbm.at[bi, pl.ds(ci * TCHUNK, TCHUNK), pl.ds(f0, FC)],
                        xbuf)

        def grp_body(g, accs):
            base = g * L
            m1v = m1buf[pl.ds(ci * TCHUNK + base, L)]
            m2v = m2buf[pl.ds(ci * TCHUNK + base, L)]
            bv1 = jnp.where(m1v > 0, 0.0, -jnp.inf).astype(jnp.float32)
            bv2 = jnp.where(m2v > 0, 0.0, -jnp.inf).astype(jnp.float32)
            new = list(accs)
            for k in range(L):
                t = base + k
                b1 = bv1[k]
                b2 = bv2[k]
                for j in range(NV):
                    v = xbuf[t, pl.ds(j * L, L)]
                    new[j] = jnp.maximum(new[j], v + b1)
                    new[NV + j] = jnp.maximum(new[NV + j], v + b2)
            return new

        return lax.fori_loop(0, TCHUNK // L, grp_body, accs)

    accs = lax.fori_loop(0, NCHUNK, chunk_body, accs)

    for j in range(NV):
        res1[pl.ds(j * L, L)] = accs[j]
        res2[pl.ds(j * L, L)] = accs[NV + j]
    pltpu.sync_copy(res1, out1_hbm.at[bi, pl.ds(f0, FC)])
    pltpu.sync_copy(res2, out2_hbm.at[bi, pl.ds(f0, FC)])


def _masked_pool_sc(x, m1i, m2i):
    f32 = jnp.float32
    return pl.kernel(
        _pool_body,
        out_type=(jax.ShapeDtypeStruct((B, K), f32),
                  jax.ShapeDtypeStruct((B, K), f32)),
        mesh=plsc.VectorSubcoreMesh(core_axis_name="c", subcore_axis_name="s",
                                    num_cores=NC, num_subcores=NS),
        compiler_params=pltpu.CompilerParams(use_tc_tiling_on_sc=False),
        scratch_types=[
            pltpu.VMEM((TCHUNK, FC), f32),
            pltpu.VMEM((S,), jnp.int32),
            pltpu.VMEM((S,), jnp.int32),
            pltpu.VMEM((FC,), f32),
            pltpu.VMEM((FC,), f32),
        ],
    )(x, m1i, m2i)


def _head_body(p1_ref, p2_ref, m1_ref, m2_ref, w_ref, b_ref, o_ref):
    c1 = jnp.sum(m1_ref[...], axis=1, keepdims=True)
    c2 = jnp.sum(m2_ref[...], axis=1, keepdims=True)
    p1 = p1_ref[...]
    p2 = p2_ref[...]
    pad1 = c1 < jnp.max(c1)
    pad2 = c2 < jnp.max(c2)
    p1 = jnp.where(pad1, jnp.maximum(p1, 0.0), p1)
    p2 = jnp.where(pad2, jnp.maximum(p2, 0.0), p2)
    dense = jnp.concatenate([p1, p2], axis=-1)
    logits = jnp.dot(dense, w_ref[...],
                     preferred_element_type=jnp.float32) + b_ref[...]
    logits = logits - jnp.max(logits, axis=-1, keepdims=True)
    e = jnp.exp(logits)
    o_ref[...] = e / jnp.sum(e, axis=-1, keepdims=True)


def _head_tc(p1, p2, m1i, m2i, W, b2d):
    return pl.pallas_call(
        _head_body,
        out_shape=jax.ShapeDtypeStruct((B, N_CLASSES), jnp.float32),
    )(p1, p2, m1i, m2i, W, b2d)


def kernel(bert_output, e1_mask, e2_mask, W, b):
    m1i = e1_mask.astype(jnp.int32)
    m2i = e2_mask.astype(jnp.int32)
    p1, p2 = _masked_pool_sc(bert_output, m1i, m2i)
    return _head_tc(p1, p2, m1i, m2i, W, b.reshape(1, N_CLASSES))


# speedup vs baseline: 3.4628x; 3.4628x over previous
import jax
import jax.numpy as jnp
from jax import lax
from jax.experimental import pallas as pl
from jax.experimental.pallas import tpu as pltpu
from jax.experimental.pallas import tpu_sc as plsc

B, S, K = 4, 2048, 768
N_CLASSES = 42
STRIP = 256
NSTRIP = S // STRIP


def _tc_body(x_ref, m1t_ref, m2t_ref, m1_ref, m2_ref, w_ref, b_ref,
             o_ref, pool_ref):
    bi = pl.program_id(0)
    si = pl.program_id(1)
    neg = jnp.float32(-jnp.inf)

    x = x_ref[0]
    b1c = m1t_ref[0]
    b2c = m2t_ref[0]
    e1 = jnp.max(x + b1c, axis=0, keepdims=True)
    e2 = jnp.max(x + b2c, axis=0, keepdims=True)
    row = pl.ds(bi, 1)

    @pl.when(si == 0)
    def _():
        pool_ref[row, 0:K] = e1
        pool_ref[row, K:2 * K] = e2

    @pl.when(si > 0)
    def _():
        pool_ref[row, 0:K] = jnp.maximum(pool_ref[row, 0:K], e1)
        pool_ref[row, K:2 * K] = jnp.maximum(pool_ref[row, K:2 * K], e2)

    @pl.when(jnp.logical_and(bi == B - 1, si == NSTRIP - 1))
    def _():
        c1 = jnp.sum(m1_ref[...], axis=1, keepdims=True)
        c2 = jnp.sum(m2_ref[...], axis=1, keepdims=True)
        pad1 = c1 < jnp.max(c1)
        pad2 = c2 < jnp.max(c2)
        p1 = pool_ref[:, 0:K]
        p2 = pool_ref[:, K:2 * K]
        p1 = jnp.where(pad1, jnp.maximum(p1, 0.0), p1)
        p2 = jnp.where(pad2, jnp.maximum(p2, 0.0), p2)
        dense = jnp.concatenate([p1, p2], axis=-1)
        logits = jnp.dot(dense, w_ref[...],
                         preferred_element_type=jnp.float32) + b_ref[...]
        logits = logits - jnp.max(logits, axis=-1, keepdims=True)
        e = jnp.exp(logits)
        o_ref[...] = e / jnp.sum(e, axis=-1, keepdims=True)


def kernel(bert_output, e1_mask, e2_mask, W, b):
    m1i = e1_mask.astype(jnp.int32)
    m2i = e2_mask.astype(jnp.int32)
    f32 = jnp.float32
    neg = jnp.float32(-jnp.inf)
    m1t = jnp.where(e1_mask, 0.0, neg).astype(f32).reshape(B, S, 1)
    m2t = jnp.where(e2_mask, 0.0, neg).astype(f32).reshape(B, S, 1)
    return pl.pallas_call(
        _tc_body,
        grid=(B, NSTRIP),
        in_specs=[
            pl.BlockSpec((1, STRIP, K), lambda bi, si: (bi, si, 0)),
            pl.BlockSpec((1, STRIP, 1), lambda bi, si: (bi, si, 0)),
            pl.BlockSpec((1, STRIP, 1), lambda bi, si: (bi, si, 0)),
            pl.BlockSpec((B, S), lambda bi, si: (0, 0)),
            pl.BlockSpec((B, S), lambda bi, si: (0, 0)),
            pl.BlockSpec((2 * K, N_CLASSES), lambda bi, si: (0, 0)),
            pl.BlockSpec((1, N_CLASSES), lambda bi, si: (0, 0)),
        ],
        out_specs=pl.BlockSpec((B, N_CLASSES), lambda bi, si: (0, 0)),
        out_shape=jax.ShapeDtypeStruct((B, N_CLASSES), f32),
        scratch_shapes=[pltpu.VMEM((B, 2 * K), f32)],
    )(bert_output, m1t, m2t, m1i, m2i, W, b.reshape(1, N_CLASSES))


# speedup vs baseline: 3.4709x; 1.0023x over previous
import jax
import jax.numpy as jnp
from jax import lax
from jax.experimental import pallas as pl
from jax.experimental.pallas import tpu as pltpu
from jax.experimental.pallas import tpu_sc as plsc

B, S, K = 4, 2048, 768
N_CLASSES = 42
STRIP = 256
NSTRIP = S // STRIP
RG = STRIP // 8


def _tc_body(x_ref, m1t_ref, m2t_ref, m1_ref, m2_ref, w_ref, b_ref,
             o_ref, acc_ref, pool_ref):
    bi = pl.program_id(0)
    si = pl.program_id(1)

    x = x_ref[0].reshape(RG, 8, K)
    b1c = m1t_ref[0].reshape(RG, 8, 1)
    b2c = m2t_ref[0].reshape(RG, 8, 1)
    e1 = jnp.max(x + b1c, axis=0)
    e2 = jnp.max(x + b2c, axis=0)
    both = jnp.concatenate([e1, e2], axis=-1)

    @pl.when(si == 0)
    def _():
        acc_ref[...] = both

    @pl.when(si > 0)
    def _():
        acc_ref[...] = jnp.maximum(acc_ref[...], both)

    @pl.when(si == NSTRIP - 1)
    def _():
        pool_ref[pl.ds(bi, 1), :] = jnp.max(acc_ref[...], axis=0,
                                            keepdims=True)

    @pl.when(jnp.logical_and(bi == B - 1, si == NSTRIP - 1))
    def _():
        c1 = jnp.sum(m1_ref[...], axis=1, keepdims=True)
        c2 = jnp.sum(m2_ref[...], axis=1, keepdims=True)
        pad1 = c1 < jnp.max(c1)
        pad2 = c2 < jnp.max(c2)
        p1 = pool_ref[:, 0:K]
        p2 = pool_ref[:, K:2 * K]
        p1 = jnp.where(pad1, jnp.maximum(p1, 0.0), p1)
        p2 = jnp.where(pad2, jnp.maximum(p2, 0.0), p2)
        dense = jnp.concatenate([p1, p2], axis=-1)
        logits = jnp.dot(dense, w_ref[...],
                         preferred_element_type=jnp.float32) + b_ref[...]
        logits = logits - jnp.max(logits, axis=-1, keepdims=True)
        e = jnp.exp(logits)
        o_ref[...] = e / jnp.sum(e, axis=-1, keepdims=True)


def kernel(bert_output, e1_mask, e2_mask, W, b):
    m1i = e1_mask.astype(jnp.int32)
    m2i = e2_mask.astype(jnp.int32)
    f32 = jnp.float32
    neg = jnp.float32(-jnp.inf)
    m1t = jnp.where(e1_mask, 0.0, neg).astype(f32).reshape(B, S, 1)
    m2t = jnp.where(e2_mask, 0.0, neg).astype(f32).reshape(B, S, 1)
    return pl.pallas_call(
        _tc_body,
        grid=(B, NSTRIP),
        in_specs=[
            pl.BlockSpec((1, STRIP, K), lambda bi, si: (bi, si, 0)),
            pl.BlockSpec((1, STRIP, 1), lambda bi, si: (bi, si, 0)),
            pl.BlockSpec((1, STRIP, 1), lambda bi, si: (bi, si, 0)),
            pl.BlockSpec((B, S), lambda bi, si: (0, 0)),
            pl.BlockSpec((B, S), lambda bi, si: (0, 0)),
            pl.BlockSpec((2 * K, N_CLASSES), lambda bi, si: (0, 0)),
            pl.BlockSpec((1, N_CLASSES), lambda bi, si: (0, 0)),
        ],
        out_specs=pl.BlockSpec((B, N_CLASSES), lambda bi, si: (0, 0)),
        out_shape=jax.ShapeDtypeStruct((B, N_CLASSES), f32),
        scratch_shapes=[pltpu.VMEM((8, 2 * K), f32),
                        pltpu.VMEM((B, 2 * K), f32)],
    )(bert_output, m1t, m2t, m1i, m2i, W, b.reshape(1, N_CLASSES))
